# baseline (speedup 1.0000x reference)
import functools

import jax
import jax.numpy as jnp
from jax import lax
from jax.experimental import pallas as pl
from jax.experimental.pallas import tpu as pltpu
from jax.experimental.pallas import tpu_sc as plsc

NC = 2
NS = 16
NW = NC * NS
L = 16

ROWS = 128
COLS = 32768
ROWS_PER_W = ROWS // NW
HALF = COLS // 2
U = 8
OUTER = HALF // (U * L)

_NEG_INF = float("-inf")
_BIG = 2**30


def _xlane(v, perm):
    return lax.gather(
        v,
        perm[:, None],
        lax.GatherDimensionNumbers(
            offset_dims=(), collapsed_slice_dims=(0,), start_index_map=(0,)
        ),
        slice_sizes=(1,),
        mode=lax.GatherScatterMode.PROMISE_IN_BOUNDS,
    )


def _merge(va, ia, vb, ib):
    take_a = (va > vb) | ((va == vb) & (ia < ib))
    return jnp.where(take_a, va, vb), jnp.where(take_a, ia, ib)


def _segment_scan(buf_ref):
    init = tuple(jnp.full((L,), _NEG_INF, jnp.float32) for _ in range(U)) + \
           tuple(jnp.zeros((L,), jnp.int32) for _ in range(U))

    def body(o, carry):
        vals = list(carry[:U])
        outs = list(carry[U:])
        o_vec = jnp.full((L,), o, jnp.int32)
        base = o * (U * L)
        for u in range(U):
            v = buf_ref[pl.ds(base + u * L, L)]
            take = v > vals[u]
            vals[u] = jnp.where(take, v, vals[u])
            outs[u] = jnp.where(take, o_vec, outs[u])
        return tuple(vals) + tuple(outs)

    res = lax.fori_loop(0, OUTER, body, init)
    return list(res[:U]), list(res[U:])


def _finalize_segment(vals, outs, seg_base, lane_iota):
    cand_v, cand_i = None, None
    for u in range(U):
        idx = outs[u] * (U * L) + (seg_base + u * L) + lane_iota
        if cand_v is None:
            cand_v, cand_i = vals[u], idx
        else:
            cand_v, cand_i = _merge(cand_v, cand_i, vals[u], idx)
    return cand_v, cand_i


def _sc_argmax_body(x_hbm, out_hbm, buf_ref, res_ref, sem0, sem1):
    wid = lax.axis_index("s") * NC + lax.axis_index("c")
    row0 = wid * ROWS_PER_W
    lane_iota = lax.iota(jnp.int32, L)
    sems = (sem0, sem1)

    def start(seg):
        r, h = divmod(seg, 2)
        b = seg % 2
        return pltpu.async_copy(
            x_hbm.at[row0 + r, pl.ds(h * HALF, HALF)], buf_ref.at[b], sems[b]
        )

    nseg = ROWS_PER_W * 2
    pending = start(0)
    res = jnp.zeros((L,), jnp.int32)

    row_v = row_i = None
    for seg in range(nseg):
        nxt = start(seg + 1) if seg + 1 < nseg else None
        pending.wait()
        pending = nxt
        r, h = divmod(seg, 2)
        vals, outs = _segment_scan(buf_ref.at[seg % 2])
        seg_v, seg_i = _finalize_segment(vals, outs, h * HALF, lane_iota)
        if h == 0:
            row_v, row_i = seg_v, seg_i
        else:
            row_v, row_i = _merge(row_v, row_i, seg_v, seg_i)
            for d in (8, 4, 2, 1):
                perm = lane_iota ^ d
                pv = _xlane(row_v, perm)
                pi = _xlane(row_i, perm)
                row_v, row_i = _merge(row_v, row_i, pv, pi)
            res = jnp.where(lane_iota == r, row_i, res)

    res_ref[...] = res
    pltpu.sync_copy(res_ref, out_hbm.at[wid])


@jax.jit
def kernel(x):
    mesh = plsc.VectorSubcoreMesh(
        core_axis_name="c", subcore_axis_name="s", num_cores=NC, num_subcores=NS
    )
    out = pl.kernel(
        _sc_argmax_body,
        out_type=jax.ShapeDtypeStruct((NW, L), jnp.int32),
        mesh=mesh,
        scratch_types=[
            pltpu.VMEM((2, HALF), jnp.float32),
            pltpu.VMEM((L,), jnp.int32),
            pltpu.SemaphoreType.DMA,
            pltpu.SemaphoreType.DMA,
        ],
    )(x)
    return out[:, :ROWS_PER_W].reshape(ROWS)
